# Initial kernel scaffold; baseline (speedup 1.0000x reference)
import jax
import jax.numpy as jnp
from jax.experimental import pallas as pl


def kernel(ts, pos_embeddings):
    raise NotImplementedError("write your pallas kernel here")


# speedup vs baseline: 6.0309x; 6.0309x over previous
import jax
import jax.numpy as jnp
from jax import lax
from jax.experimental import pallas as pl

BB = 16


def _posenc_block(ts_ref, emb_ref, out_ref):
    t = ts_ref[...]
    bb, hist = t.shape
    expand = emb_ref.shape[1]
    tk = t[:, :, None]
    tj = t[:, None, :]
    kk = lax.broadcasted_iota(jnp.int32, (bb, hist, hist), 1)
    jj = lax.broadcasted_iota(jnp.int32, (bb, hist, hist), 2)
    c = ((tk < tj) | ((tk <= tj) & (kk < jj))).astype(jnp.int32)
    rank = jnp.sum(c, axis=1)
    ii = lax.broadcasted_iota(jnp.int32, (bb, hist, hist), 1)
    m = (rank[:, None, :] == ii).astype(jnp.float32)
    out = jnp.dot(m.reshape(bb * hist, hist), emb_ref[...],
                  preferred_element_type=jnp.float32)
    out_ref[...] = out.reshape(bb, hist, expand)


def kernel(ts, pos_embeddings):
    batch, hist = ts.shape
    seq_len, expand = pos_embeddings.shape
    return pl.pallas_call(
        _posenc_block,
        grid=(batch // BB,),
        in_specs=[
            pl.BlockSpec((BB, hist), lambda i: (i, 0)),
            pl.BlockSpec((seq_len, expand), lambda i: (0, 0)),
        ],
        out_specs=pl.BlockSpec((BB, hist, expand), lambda i: (i, 0, 0)),
        out_shape=jax.ShapeDtypeStruct((batch, hist, expand), jnp.float32),
    )(ts, pos_embeddings)
